# baseline (speedup 1.0000x reference)
import functools

import jax
import jax.numpy as jnp
from jax import lax
from jax.experimental import pallas as pl
from jax.experimental.pallas import tpu as pltpu
from jax.experimental.pallas import tpu_sc as plsc

D = 64
CHUNK = 128
LANES = 16


def _make_body(n_flat, seq, n_workers):
  per_w = n_flat // n_workers
  n_chunks = per_w // CHUNK
  assert per_w % CHUNK == 0
  assert per_w % seq == 0

  mesh = plsc.VectorSubcoreMesh(core_axis_name="c", subcore_axis_name="s")

  @functools.partial(
      pl.kernel,
      out_type=jax.ShapeDtypeStruct((n_flat, D), jnp.float32),
      mesh=mesh,
      compiler_params=pltpu.CompilerParams(use_tc_tiling_on_sc=False),
      scratch_types=[
          pltpu.VMEM((CHUNK,), jnp.int32),
          pltpu.VMEM((CHUNK, D), jnp.float32),
          pltpu.VMEM((2 * seq, D), jnp.float32),
          pltpu.SemaphoreType.DMA,
      ],
  )
  def body(idx_hbm, table_hbm, pos_hbm, out_hbm, idx_v, rows_v, pe2_v, sem):
    nc = lax.axis_size("c")
    wid = lax.axis_index("s") * nc + lax.axis_index("c")
    base = wid * per_w

    pltpu.sync_copy(pos_hbm.at[pl.ds(0, seq)], pe2_v.at[pl.ds(0, seq)])
    pltpu.sync_copy(pos_hbm.at[pl.ds(0, seq)], pe2_v.at[pl.ds(seq, seq)])

    def chunk_body(c, _):
      g = base + c * CHUNK
      phase = lax.rem(c * CHUNK, seq)
      pltpu.sync_copy(idx_hbm.at[pl.ds(g, CHUNK)], idx_v)
      pltpu.async_copy(table_hbm.at[idx_v], rows_v, sem).wait()

      def add_body(i, _):
        for j in range(D // LANES):
          sl = pl.ds(j * LANES, LANES)
          plsc.addupdate(rows_v.at[i, sl], pe2_v[phase + i, sl])
        return _

      lax.fori_loop(0, CHUNK, add_body, None, unroll=False)
      pltpu.sync_copy(rows_v, out_hbm.at[pl.ds(g, CHUNK)])
      return _

    lax.fori_loop(0, n_chunks, chunk_body, None, unroll=False)

  return body


def kernel(inputs, table, pos_encoding):
  b, s = inputs.shape
  n_flat = b * s
  info = plsc.get_sparse_core_info()
  n_workers = info.num_cores * info.num_subcores
  flat_idx = inputs.reshape(n_flat).astype(jnp.int32)
  body = _make_body(n_flat, s, n_workers)
  out = body(flat_idx, table, pos_encoding)
  return out.reshape(b, s, D)


# speedup vs baseline: 1.4772x; 1.4772x over previous
import functools

import jax
import jax.numpy as jnp
from jax import lax
from jax.experimental import pallas as pl
from jax.experimental.pallas import tpu as pltpu
from jax.experimental.pallas import tpu_sc as plsc

D = 64
LANES = 16
GSUB = 100
SEQ_PER_IT = 2


def _make_body(n_batch, seq, n_cores, n_subcores):
  n_workers = n_cores * n_subcores
  assert n_batch % (n_workers * SEQ_PER_IT) == 0
  assert seq % GSUB == 0
  b_per_w = n_batch // n_workers
  n_it = b_per_w // SEQ_PER_IT
  assert n_it % 2 == 0
  rows_per_it = SEQ_PER_IT * seq
  g_per_it = rows_per_it // GSUB
  idx_rows_w = b_per_w * seq // GSUB

  mesh = plsc.VectorSubcoreMesh(core_axis_name="c", subcore_axis_name="s")

  @functools.partial(
      pl.kernel,
      out_type=jax.ShapeDtypeStruct((n_batch, seq, D), jnp.float32),
      mesh=mesh,
      compiler_params=pltpu.CompilerParams(use_tc_tiling_on_sc=False),
      scratch_types=[
          pltpu.VMEM((idx_rows_w, GSUB), jnp.int32),
          pltpu.VMEM((SEQ_PER_IT, seq, D), jnp.float32),
          pltpu.VMEM((SEQ_PER_IT, seq, D), jnp.float32),
          pltpu.VMEM((seq, D), jnp.float32),
          pltpu.SemaphoreType.DMA,
          pltpu.SemaphoreType.DMA,
          pltpu.SemaphoreType.DMA,
          pltpu.SemaphoreType.DMA,
      ],
  )
  def body(idx_hbm, table_hbm, pos_hbm, out_hbm,
           idx_v, rows0, rows1, pe_v,
           sem_g0, sem_g1, sem_w0, sem_w1):
    rows = (rows0, rows1)
    sem_g = (sem_g0, sem_g1)
    sem_w = (sem_w0, sem_w1)

    wid = lax.axis_index("s") * n_cores + lax.axis_index("c")
    seq0 = wid * b_per_w
    idx_row0 = wid * idx_rows_w

    pltpu.sync_copy(idx_hbm.at[pl.ds(idx_row0, idx_rows_w)], idx_v)
    pltpu.sync_copy(pos_hbm.at[pl.ds(0, seq)], pe_v)

    def fire_gathers(t, b):
      for j in range(g_per_it):
        src = table_hbm.at[idx_v.at[t * g_per_it + j]]
        dst = rows[b].at[j // 2, pl.ds((j % 2) * GSUB, GSUB)]
        pltpu.async_copy(src, dst, sem_g[b])

    def drain_gathers(t, b):
      for j in range(g_per_it):
        src = table_hbm.at[idx_v.at[t * g_per_it + j]]
        dst = rows[b].at[j // 2, pl.ds((j % 2) * GSUB, GSUB)]
        pltpu.make_async_copy(src, dst, sem_g[b]).wait()

    def fire_write(t, b):
      pltpu.async_copy(rows[b], out_hbm.at[pl.ds(seq0 + t * SEQ_PER_IT,
                                                 SEQ_PER_IT)], sem_w[b])

    def drain_write(t, b):
      pltpu.make_async_copy(rows[b], out_hbm.at[pl.ds(seq0 + t * SEQ_PER_IT,
                                                      SEQ_PER_IT)],
                            sem_w[b]).wait()

    def add_pe(b):
      def add_body(i, carry):
        for k in range(SEQ_PER_IT):
          for j in range(D // LANES):
            sl = pl.ds(j * LANES, LANES)
            plsc.addupdate(rows[b].at[k, i, sl], pe_v[i, sl])
        return carry
      lax.fori_loop(0, seq, add_body, None, unroll=False)

    fire_gathers(0, 0)

    def step(t2, carry):
      t = 2 * t2

      @pl.when(t2 > 0)
      def _():
        drain_write(t - 1, 1)
      fire_gathers(t + 1, 1)
      drain_gathers(t, 0)
      add_pe(0)
      fire_write(t, 0)

      t = 2 * t2 + 1
      drain_write(t - 1, 0)

      @pl.when(t2 < n_it // 2 - 1)
      def _():
        fire_gathers(t + 1, 0)
      drain_gathers(t, 1)
      add_pe(1)
      fire_write(t, 1)
      return carry

    lax.fori_loop(0, n_it // 2, step, None, unroll=False)
    drain_write(n_it - 1, 1)

  return body


def kernel(inputs, table, pos_encoding):
  b, s = inputs.shape
  info = plsc.get_sparse_core_info()
  idx2d = inputs.reshape(b * s // GSUB, GSUB).astype(jnp.int32)
  body = _make_body(b, s, info.num_cores, info.num_subcores)
  return body(idx2d, table, pos_encoding)
